# Initial kernel scaffold; baseline (speedup 1.0000x reference)
import jax
import jax.numpy as jnp
from jax.experimental import pallas as pl


def kernel(h, idx):
    raise NotImplementedError("write your pallas kernel here")


# speedup vs baseline: 3.4666x; 3.4666x over previous
import functools

import jax
import jax.numpy as jnp
from jax import lax
from jax.experimental import pallas as pl
from jax.experimental.pallas import tpu as pltpu
from jax.experimental.pallas import tpu_sc as plsc

_R = 2.0
_T = 1.0
_NC = 2
_NS = 16
_NW = _NC * _NS
_C = 80
_L = 16
_DIM = 128


def _normalize_body(h_ref, o_ref):
    h = h_ref[...]
    ss = jnp.sum(h * h, axis=1, keepdims=True)
    norm = jnp.sqrt(ss)
    scale = jnp.minimum(1.0, 1.0 / jnp.maximum(norm, 1e-12))
    o_ref[...] = h * scale


def _normalize(h):
    return pl.pallas_call(
        _normalize_body,
        out_shape=jax.ShapeDtypeStruct(h.shape, h.dtype),
    )(h)


@functools.lru_cache(maxsize=None)
def _sc_decode(n_edges):
    ew = n_edges // _NW
    nchunks = ew // _C
    mesh = plsc.VectorSubcoreMesh(core_axis_name="c", subcore_axis_name="s")

    @functools.partial(
        pl.kernel,
        out_type=jax.ShapeDtypeStruct((n_edges,), jnp.float32),
        mesh=mesh,
        compiler_params=pltpu.CompilerParams(needs_layout_passes=False),
        scratch_types=[
            pltpu.VMEM((_C,), jnp.int32),
            pltpu.VMEM((_C,), jnp.int32),
            pltpu.VMEM((_C, _DIM), jnp.float32),
            pltpu.VMEM((_C, _DIM), jnp.float32),
            pltpu.VMEM((_C, _L), jnp.float32),
            pltpu.VMEM((_C,), jnp.float32),
            pltpu.SemaphoreType.DMA,
            pltpu.SemaphoreType.DMA,
        ],
    )
    def k(tab_hbm, idx0_hbm, idx1_hbm, out_hbm,
          idx0_v, idx1_v, rows_a, rows_b, sq, outv, sem_a, sem_b):
        wid = lax.axis_index("s") * _NC + lax.axis_index("c")
        base0 = wid * ew

        def chunk_body(g, carry):
            base = base0 + g * _C
            pltpu.sync_copy(idx0_hbm.at[pl.ds(base, _C)], idx0_v)
            pltpu.sync_copy(idx1_hbm.at[pl.ds(base, _C)], idx1_v)
            cp_a = pltpu.async_copy(tab_hbm.at[idx0_v], rows_a, sem_a)
            cp_b = pltpu.async_copy(tab_hbm.at[idx1_v], rows_b, sem_b)
            cp_a.wait()
            cp_b.wait()

            def edge_body(e, c2):
                acc = jnp.zeros((_L,), jnp.float32)
                for kk in range(_DIM // _L):
                    va = rows_a[e, pl.ds(kk * _L, _L)]
                    vb = rows_b[e, pl.ds(kk * _L, _L)]
                    d = va - vb
                    acc = acc + d * d
                sq[e, :] = acc
                return c2

            lax.fori_loop(0, _C, edge_body, 0)

            def grp_body(gg, c2):
                e0 = gg * _L
                rows = e0 + lax.iota(jnp.int32, _L)
                tot = jnp.zeros((_L,), jnp.float32)
                for col in range(_L):
                    cols = jnp.full((_L,), col, jnp.int32)
                    tot = tot + plsc.load_gather(sq, [rows, cols])
                p = 1.0 / (1.0 + jnp.exp((tot - _R) / _T))
                outv[pl.ds(e0, _L)] = p
                return c2

            lax.fori_loop(0, _C // _L, grp_body, 0)

            pltpu.sync_copy(outv, out_hbm.at[pl.ds(base, _C)])
            return carry

        lax.fori_loop(0, nchunks, chunk_body, 0)

    return k


def kernel(h, idx):
    hn = _normalize(h)
    idx0 = idx[:, 0]
    idx1 = idx[:, 1]
    return _sc_decode(idx.shape[0])(hn, idx0, idx1)


# speedup vs baseline: 7.5567x; 2.1799x over previous
import functools

import jax
import jax.numpy as jnp
from jax import lax
from jax.experimental import pallas as pl
from jax.experimental.pallas import tpu as pltpu
from jax.experimental.pallas import tpu_sc as plsc

_R = 2.0
_T = 1.0
_NC = 2
_NS = 16
_NW = _NC * _NS
_C = 80
_L = 16
_DIM = 128


def _normalize_body(h_ref, o_ref):
    h = h_ref[...]
    ss = jnp.sum(h * h, axis=1, keepdims=True)
    norm = jnp.sqrt(ss)
    scale = jnp.minimum(1.0, 1.0 / jnp.maximum(norm, 1e-12))
    o_ref[...] = h * scale


def _normalize(h):
    return pl.pallas_call(
        _normalize_body,
        out_shape=jax.ShapeDtypeStruct(h.shape, h.dtype),
    )(h)


@functools.lru_cache(maxsize=None)
def _sc_decode(n_edges):
    ew = n_edges // _NW
    nchunks = ew // _C
    mesh = plsc.VectorSubcoreMesh(core_axis_name="c", subcore_axis_name="s")

    @functools.partial(
        pl.kernel,
        out_type=jax.ShapeDtypeStruct((n_edges,), jnp.float32),
        mesh=mesh,
        compiler_params=pltpu.CompilerParams(needs_layout_passes=False),
        scratch_types=[
            pltpu.VMEM((ew,), jnp.int32),
            pltpu.VMEM((ew,), jnp.int32),
            pltpu.VMEM((2, _C, _DIM), jnp.float32),
            pltpu.VMEM((2, _C, _DIM), jnp.float32),
            pltpu.VMEM((_C, _L), jnp.float32),
            pltpu.VMEM((ew,), jnp.float32),
            pltpu.SemaphoreType.DMA,
            pltpu.SemaphoreType.DMA,
            pltpu.SemaphoreType.DMA,
            pltpu.SemaphoreType.DMA,
            pltpu.SemaphoreType.DMA,
        ],
    )
    def k(tab_hbm, idx0_hbm, idx1_hbm, out_hbm,
          idx0_w, idx1_w, rows_a, rows_b, sq, out_w,
          sem_i, sem_a0, sem_a1, sem_b0, sem_b1):
        wid = lax.axis_index("s") * _NC + lax.axis_index("c")
        base0 = wid * ew
        sems_a = (sem_a0, sem_a1)
        sems_b = (sem_b0, sem_b1)

        pltpu.async_copy(idx0_hbm.at[pl.ds(base0, ew)], idx0_w, sem_i)
        pltpu.async_copy(idx1_hbm.at[pl.ds(base0, ew)], idx1_w, sem_i)
        pltpu.make_async_copy(idx0_hbm.at[pl.ds(base0, ew)], idx0_w, sem_i).wait()
        pltpu.make_async_copy(idx1_hbm.at[pl.ds(base0, ew)], idx1_w, sem_i).wait()

        def start(g, b):
            pltpu.async_copy(
                tab_hbm.at[idx0_w.at[pl.ds(g * _C, _C)]], rows_a.at[b], sems_a[b])
            pltpu.async_copy(
                tab_hbm.at[idx1_w.at[pl.ds(g * _C, _C)]], rows_b.at[b], sems_b[b])

        def wait(b):
            pltpu.make_async_copy(
                tab_hbm.at[idx0_w.at[pl.ds(0, _C)]], rows_a.at[b], sems_a[b]).wait()
            pltpu.make_async_copy(
                tab_hbm.at[idx1_w.at[pl.ds(0, _C)]], rows_b.at[b], sems_b[b]).wait()

        def compute(g, b):
            @plsc.parallel_loop(0, _C, 1, unroll=4)
            def _(e):
                acc = jnp.zeros((_L,), jnp.float32)
                for kk in range(_DIM // _L):
                    va = rows_a[b, e, pl.ds(kk * _L, _L)]
                    vb = rows_b[b, e, pl.ds(kk * _L, _L)]
                    d = va - vb
                    acc = acc + d * d
                sq[e, :] = acc

            e0g = g * _C
            for gg in range(_C // _L):
                e0 = gg * _L
                rows = e0 + lax.iota(jnp.int32, _L)
                tot = jnp.zeros((_L,), jnp.float32)
                for col in range(_L):
                    cols = jnp.full((_L,), col, jnp.int32)
                    tot = tot + plsc.load_gather(sq, [rows, cols])
                p = 1.0 / (1.0 + jnp.exp((tot - _R) / _T))
                out_w[pl.ds(e0g + e0, _L)] = p

        start(0, 0)
        start(1, 1)

        def pair_body(g2, carry):
            wait(0)
            compute(2 * g2, 0)
            start(2 * g2 + 2, 0)

            wait(1)
            compute(2 * g2 + 1, 1)

            @pl.when(g2 < nchunks // 2 - 1)
            def _():
                start(2 * g2 + 3, 1)

            return carry

        lax.fori_loop(0, nchunks // 2, pair_body, 0)

        if nchunks % 2 == 1:
            wait(0)
            compute(nchunks - 1, 0)

        pltpu.sync_copy(out_w, out_hbm.at[pl.ds(base0, ew)])

    return k


def kernel(h, idx):
    hn = _normalize(h)
    idx0 = idx[:, 0]
    idx1 = idx[:, 1]
    return _sc_decode(idx.shape[0])(hn, idx0, idx1)
